# baseline (speedup 1.0000x reference)
import functools
import math

import jax
import jax.numpy as jnp
from jax import lax
from jax.experimental import pallas as pl
from jax.experimental.pallas import tpu as pltpu
from jax.experimental.pallas import tpu_sc as plsc

NC = 2
NS = 16
NW = NC * NS
LANES = 16

GATHER_ROWS = 128


def _make_pe(seq_len: int, d: int) -> jax.Array:
    pos = jnp.arange(0, seq_len, dtype=jnp.float32)[:, None]
    fill = pos * jnp.exp(
        -jnp.arange(0, d, 2, dtype=jnp.float32) * math.log(10000.0) / d
    )
    pe = jnp.zeros((seq_len, d), dtype=jnp.float32)
    pe = pe.at[:, 0::2].set(jnp.sin(fill))
    pe = pe.at[:, 1::2].set(jnp.cos(fill))
    return pe


@functools.partial(jax.jit, static_argnames=("n_rows", "d", "seq_len"))
def _sc_embed(x2d, pe, table, *, n_rows, d, seq_len):
    per_w = n_rows // NW
    n_g = per_w // GATHER_ROWS
    half = per_w // 2
    g_per_half = n_g // 2
    vecs_per_row = d // LANES

    mesh = plsc.VectorSubcoreMesh(core_axis_name="c", subcore_axis_name="s")

    @functools.partial(
        pl.kernel,
        out_type=jax.ShapeDtypeStruct((n_rows, d), jnp.float32),
        mesh=mesh,
        compiler_params=pltpu.CompilerParams(use_tc_tiling_on_sc=False),
        scratch_types=[
            pltpu.VMEM((n_g, GATHER_ROWS), jnp.int32),
            pltpu.VMEM((half, d), jnp.float32),
            pltpu.VMEM((half, d), jnp.float32),
            pltpu.SemaphoreType.DMA,
        ],
    )
    def body(x_hbm, pe_hbm, table_hbm, out_hbm, idx_v, rows_v, pe_v, sem):
        wid = lax.axis_index("s") * NC + lax.axis_index("c")
        base = wid * per_w
        l_start = lax.rem(base, seq_len)

        pltpu.sync_copy(x_hbm.at[pl.ds(wid * n_g, n_g)], idx_v)

        for c in range(2):
            copies = [
                pltpu.async_copy(
                    table_hbm.at[idx_v.at[c * g_per_half + k]],
                    rows_v.at[pl.ds(k * GATHER_ROWS, GATHER_ROWS)],
                    sem,
                )
                for k in range(g_per_half)
            ]
            pltpu.sync_copy(pe_hbm.at[pl.ds(l_start + c * half, half)], pe_v)
            for cp in copies:
                cp.wait()

            def row_add(i, carry):
                for j in range(vecs_per_row):
                    sl = pl.ds(j * LANES, LANES)
                    plsc.addupdate(rows_v.at[i, sl], pe_v[i, sl])
                return carry

            lax.fori_loop(0, half, row_add, 0)
            pltpu.sync_copy(rows_v, out_hbm.at[pl.ds(base + c * half, half)])

    return body(x2d, pe, table)


def kernel(x, table):
    b, l = x.shape
    v, d = table.shape
    n_rows = b * l
    pe = _make_pe(l, d)
    x2d = x.reshape(n_rows // GATHER_ROWS, GATHER_ROWS).astype(jnp.int32)
    out = _sc_embed(x2d, pe, table, n_rows=n_rows, d=d, seq_len=l)
    return out.reshape(b, l, d)
